# baseline (speedup 1.0000x reference)
import functools

import jax
import jax.numpy as jnp
from jax import lax
from jax.experimental import pallas as pl
from jax.experimental.pallas import tpu as pltpu
from jax.experimental.pallas import tpu_sc as plsc

N = 10000
E = 320000
HID = 128
NT = 4
N_STEPS = 5

NC = 2
NS = 16
NW = NC * NS

K = 24
Q = 2
HK = Q * K
NH = 210
CHH = NH + 2
BUCK = 640
N_SH = NS * BUCK
ACC_ROWS = N_SH + 8
ZR = 64


@functools.cache
def _get_sc_aggregate():
    mesh = plsc.VectorSubcoreMesh(core_axis_name="c", subcore_axis_name="s")

    scratch = (
        [pltpu.VMEM((2, HK), jnp.int32),
         pltpu.VMEM((2, HK), jnp.int32),
         pltpu.VMEM((2, HK, HID), jnp.float32),
         pltpu.VMEM((ZR, HID), jnp.float32),
         pltpu.VMEM_SHARED((ACC_ROWS, HID), jnp.float32)]
        + [pltpu.SemaphoreType.DMA for _ in range(2 * Q + 4)]
    )

    @functools.partial(
        pl.kernel,
        out_type=jax.ShapeDtypeStruct((NC, N_SH, HID), jnp.float32),
        mesh=mesh,
        scratch_types=scratch,
    )
    def _sc_aggregate(tbl_hbm, eidx_hbm, dst_hbm, out_hbm,
                      eix, dsx, rbuf, zbuf, acc, *sems):
        sems_g = sems[0:2 * Q]
        sems_e = sems[2 * Q:2 * Q + 2]
        sems_d = sems[2 * Q + 2:]

        cid = lax.axis_index("c")
        sid = lax.axis_index("s")
        wid = sid * NC + cid

        def _zero_row(r, carry):
            for j in range(HID // 16):
                zbuf[r, pl.ds(j * 16, 16)] = jnp.zeros((16,), jnp.float32)
            return carry

        lax.fori_loop(0, ZR, _zero_row, 0)
        for q in range(BUCK // ZR):
            pltpu.sync_copy(zbuf, acc.at[pl.ds(sid * BUCK + q * ZR, ZR)])
        plsc.subcore_barrier()

        def idx_cps(t, h):
            return (pltpu.make_async_copy(eidx_hbm.at[wid, t], eix.at[h],
                                          sems_e[h]),
                    pltpu.make_async_copy(dst_hbm.at[wid, t], dsx.at[h],
                                          sems_d[h]))

        def gat_cp(h, j):
            return pltpu.make_async_copy(
                tbl_hbm.at[eix.at[h, pl.ds(j * K, K)]],
                rbuf.at[h, pl.ds(j * K, K)],
                sems_g[h * Q + j])

        for cp in idx_cps(0, 0):
            cp.start()
        for cp in idx_cps(0, 0):
            cp.wait()
        for j in range(Q):
            gat_cp(0, j).start()
        for cp in idx_cps(1, 1):
            cp.start()

        def _pair(i, carry):
            t0 = 2 * i
            for h in (0, 1):
                t = t0 + h
                for j in range(Q):
                    gat_cp(h, j).wait()
                for cp in idx_cps(t + 1, 1 - h):
                    cp.wait()
                for j in range(Q):
                    gat_cp(1 - h, j).start()
                pltpu.sync_copy(rbuf.at[h], acc.at[dsx.at[h]], add=True)
                for cp in idx_cps(t + 2, h):
                    cp.start()
            return carry

        lax.fori_loop(0, NH // 2, _pair, 0)

        for j in range(Q):
            gat_cp(0, j).wait()
        for cp in idx_cps(NH + 1, 1):
            cp.wait()

        plsc.subcore_barrier()
        pltpu.sync_copy(
            acc.at[pl.ds(sid * BUCK, BUCK)],
            out_hbm.at[cid, pl.ds(sid * BUCK, BUCK)],
        )

    return _sc_aggregate


BN = 2000


def _prologue_body(h_ref, winT_ref, bin_ref, wcat_ref, bcat_ref,
                   x_ref, tbl_ref):
    x = jnp.dot(h_ref[...], winT_ref[...],
                preferred_element_type=jnp.float32) + bin_ref[...]
    x_ref[...] = x
    tbl_ref[...] = jnp.dot(x, wcat_ref[...],
                           preferred_element_type=jnp.float32) + bcat_ref[...]


def _gru_math(ap_ref, x_ref, wihT_ref, bih_ref, whhT_ref, bhh_ref):
    a = ap_ref[0] + ap_ref[1]
    x = x_ref[...]
    gi = jnp.dot(a, wihT_ref[...], preferred_element_type=jnp.float32) + bih_ref[...]
    gh = jnp.dot(x, whhT_ref[...], preferred_element_type=jnp.float32) + bhh_ref[...]
    r = jax.nn.sigmoid(gi[:, :HID] + gh[:, :HID])
    z = jax.nn.sigmoid(gi[:, HID:2 * HID] + gh[:, HID:2 * HID])
    nh = jnp.tanh(gi[:, 2 * HID:] + r * gh[:, 2 * HID:])
    return (1.0 - z) * nh + z * x


def _gru_body(ap_ref, x_ref, wihT_ref, bih_ref, whhT_ref, bhh_ref,
              wcat_ref, bcat_ref, xo_ref, tbl_ref):
    xn = _gru_math(ap_ref, x_ref, wihT_ref, bih_ref, whhT_ref, bhh_ref)
    xo_ref[...] = xn
    tbl_ref[...] = jnp.dot(xn, wcat_ref[...],
                           preferred_element_type=jnp.float32) + bcat_ref[...]


def _gru_last_body(ap_ref, x_ref, wihT_ref, bih_ref, whhT_ref, bhh_ref,
                   w1T_ref, b1_ref, w2T_ref, b2_ref, out_ref, acc_ref):
    xn = _gru_math(ap_ref, x_ref, wihT_ref, bih_ref, whhT_ref, bhh_ref)
    ps = jnp.sum(xn, axis=0, keepdims=True)
    i = pl.program_id(0)

    @pl.when(i == 0)
    def _():
        acc_ref[...] = ps

    @pl.when(i > 0)
    def _():
        acc_ref[...] += ps

    @pl.when(i == (N // BN) - 1)
    def _():
        hg = acc_ref[...] * (1.0 / N)
        h1 = jnp.maximum(
            jnp.dot(hg, w1T_ref[...], preferred_element_type=jnp.float32)
            + b1_ref[...], 0.0)
        out_ref[...] = jnp.dot(h1, w2T_ref[...],
                               preferred_element_type=jnp.float32) + b2_ref[...]


def _row_spec(cols):
    return pl.BlockSpec((BN, cols), lambda i: (i, 0))


def _full_spec(rows, cols):
    return pl.BlockSpec((rows, cols), lambda i: (0, 0))


_GRID = (N // BN,)


def _prologue(h, W_inT, b_in2, Wcat, bcat):
    return pl.pallas_call(
        _prologue_body,
        grid=_GRID,
        in_specs=[_row_spec(HID), _full_spec(HID, HID), _full_spec(1, HID),
                  _full_spec(HID, NT * HID), _full_spec(1, NT * HID)],
        out_specs=[_row_spec(HID), _row_spec(NT * HID)],
        out_shape=[jax.ShapeDtypeStruct((N, HID), jnp.float32),
                   jax.ShapeDtypeStruct((N, NT * HID), jnp.float32)],
    )(h, W_inT, b_in2, Wcat, bcat)


_AP_SPEC = pl.BlockSpec((NC, BN, HID), lambda i: (0, i, 0))


def _gru_step(ap, x, w_ihT, b_ih2, w_hhT, b_hh2, Wcat, bcat):
    return pl.pallas_call(
        _gru_body,
        grid=_GRID,
        in_specs=[_AP_SPEC, _row_spec(HID),
                  _full_spec(HID, 3 * HID), _full_spec(1, 3 * HID),
                  _full_spec(HID, 3 * HID), _full_spec(1, 3 * HID),
                  _full_spec(HID, NT * HID), _full_spec(1, NT * HID)],
        out_specs=[_row_spec(HID), _row_spec(NT * HID)],
        out_shape=[jax.ShapeDtypeStruct((N, HID), jnp.float32),
                   jax.ShapeDtypeStruct((N, NT * HID), jnp.float32)],
    )(ap, x, w_ihT, b_ih2, w_hhT, b_hh2, Wcat, bcat)


def _gru_last(ap, x, w_ihT, b_ih2, w_hhT, b_hh2, W1T, b1r, W2T, b2r):
    return pl.pallas_call(
        _gru_last_body,
        grid=_GRID,
        in_specs=[_AP_SPEC, _row_spec(HID),
                  _full_spec(HID, 3 * HID), _full_spec(1, 3 * HID),
                  _full_spec(HID, 3 * HID), _full_spec(1, 3 * HID),
                  _full_spec(HID, HID // 2), _full_spec(1, HID // 2),
                  _full_spec(HID // 2, 10), _full_spec(1, 10)],
        out_specs=pl.BlockSpec((1, 10), lambda i: (0, 0)),
        out_shape=jax.ShapeDtypeStruct((1, 10), jnp.float32),
        scratch_shapes=[pltpu.VMEM((1, HID), jnp.float32)],
    )(ap, x, w_ihT, b_ih2, w_hhT, b_hh2, W1T, b1r, W2T, b2r)


def kernel(h, edge_index, etypes, W_in, b_in, W_et, b_et,
           w_ih, w_hh, b_ih, b_hh, W1, b1, W2, b2):
    src = edge_index[0]
    dst = edge_index[1]
    eidx = src * NT + etypes
    pad = CHH * HK - E // NW
    eidx_p = jnp.concatenate(
        [eidx.reshape(NW, E // NW), jnp.zeros((NW, pad), jnp.int32)],
        axis=1).reshape(NW, CHH, HK)
    dst_p = jnp.concatenate(
        [dst.reshape(NW, E // NW), jnp.full((NW, pad), N_SH, jnp.int32)],
        axis=1).reshape(NW, CHH, HK)

    W_inT = W_in.T
    b_in2 = b_in.reshape(1, HID)
    Wcat = jnp.transpose(W_et, (2, 0, 1)).reshape(HID, NT * HID)
    bcat = b_et.reshape(1, NT * HID)
    w_ihT = w_ih.T
    w_hhT = w_hh.T
    b_ih2 = b_ih.reshape(1, 3 * HID)
    b_hh2 = b_hh.reshape(1, 3 * HID)
    W1T = W1.T
    b1r = b1.reshape(1, HID // 2)
    W2T = W2.T
    b2r = b2.reshape(1, 10)

    x, tbl = _prologue(h, W_inT, b_in2, Wcat, bcat)
    sc_aggregate = _get_sc_aggregate()
    for step in range(N_STEPS):
        ap = sc_aggregate(tbl.reshape(N * NT, HID), eidx_p, dst_p)
        if step < N_STEPS - 1:
            x, tbl = _gru_step(ap, x, w_ihT, b_ih2, w_hhT, b_hh2, Wcat, bcat)
        else:
            out = _gru_last(ap, x, w_ihT, b_ih2, w_hhT, b_hh2,
                            W1T, b1r, W2T, b2r)
    return out
